# baseline (speedup 1.0000x reference)
import functools

import jax
import jax.numpy as jnp
from jax import lax
from jax.experimental import pallas as pl
from jax.experimental.pallas import tpu as pltpu
from jax.experimental.pallas import tpu_sc as plsc

B = 16384
K = 32
N = 1000000
NC = 2
NS = 16
NW = NC * NS
BPW = B // NW
CI = 64
NCH = BPW // CI
LANES = 16
GPC = CI // LANES
NSLOT = 2

M_OFF = N * K // 128
UB_OFF = 2 * N * K // 128
MB_BASE = 2 * N * K + N


def _body(big, xi, out,
          uf_v, mf_v, urow_v, mrow_v, ubrow_v, mbrow_v,
          ubuf, mbuf, ubbuf, mbbuf, out_v, sems):
    wid = lax.axis_index("s") * NC + lax.axis_index("c")

    pltpu.sync_copy(xi.at[0, wid], uf_v)
    pltpu.sync_copy(xi.at[1, wid], mf_v)

    for t in range(BPW // LANES):
        o = t * LANES
        u = uf_v[pl.ds(o, LANES)]
        m = mf_v[pl.ds(o, LANES)]
        urow_v[pl.ds(o, LANES)] = u >> 2
        mrow_v[pl.ds(o, LANES)] = M_OFF + (m >> 2)
        ubrow_v[pl.ds(o, LANES)] = UB_OFF + (u >> 7)
        mbrow_v[pl.ds(o, LANES)] = (MB_BASE + m) >> 7

    def fire(j, s):
        sl = pl.ds(j * CI, CI)
        pltpu.async_copy(big.at[urow_v.at[sl]], ubuf.at[s], sems.at[s])
        pltpu.async_copy(big.at[mrow_v.at[sl]], mbuf.at[s], sems.at[s])
        pltpu.async_copy(big.at[ubrow_v.at[sl]], ubbuf.at[s], sems.at[s])
        pltpu.async_copy(big.at[mbrow_v.at[sl]], mbbuf.at[s], sems.at[s])

    def drain(j, s):
        sl = pl.ds(j * CI, CI)
        pltpu.make_async_copy(big.at[urow_v.at[sl]], ubuf.at[s], sems.at[s]).wait()
        pltpu.make_async_copy(big.at[mrow_v.at[sl]], mbuf.at[s], sems.at[s]).wait()
        pltpu.make_async_copy(big.at[ubrow_v.at[sl]], ubbuf.at[s], sems.at[s]).wait()
        pltpu.make_async_copy(big.at[mbrow_v.at[sl]], mbbuf.at[s], sems.at[s]).wait()

    fire(0, 0)
    for j in range(NCH):
        s = j % NSLOT
        drain(j, s)
        if j + 1 < NCH:
            fire(j + 1, (j + 1) % NSLOT)
        for g in range(GPC):
            o = j * CI + g * LANES
            u = uf_v[pl.ds(o, LANES)]
            m = mf_v[pl.ds(o, LANES)]
            lane = g * LANES + lax.iota(jnp.int32, LANES)
            offu = (u & 3) * 32
            offm = (m & 3) * 32
            acc = (plsc.load_gather(ubbuf.at[s], [lane, u & 127]) +
                   plsc.load_gather(mbbuf.at[s], [lane, (MB_BASE + m) & 127]))
            for k in range(K):
                gu = plsc.load_gather(ubuf.at[s], [lane, offu + k])
                gm = plsc.load_gather(mbuf.at[s], [lane, offm + k])
                acc = acc + gu * gm
            out_v[pl.ds(o, LANES)] = acc

    pltpu.sync_copy(out_v, out.at[pl.ds(wid * BPW, BPW)])


_run = functools.partial(
    pl.kernel,
    out_type=jax.ShapeDtypeStruct((B,), jnp.float32),
    mesh=plsc.VectorSubcoreMesh(core_axis_name="c", subcore_axis_name="s"),
    compiler_params=pltpu.CompilerParams(needs_layout_passes=False),
    scratch_types=[
        pltpu.VMEM((BPW,), jnp.int32),
        pltpu.VMEM((BPW,), jnp.int32),
        pltpu.VMEM((BPW,), jnp.int32),
        pltpu.VMEM((BPW,), jnp.int32),
        pltpu.VMEM((BPW,), jnp.int32),
        pltpu.VMEM((BPW,), jnp.int32),
        pltpu.VMEM((NSLOT, CI, 128), jnp.float32),
        pltpu.VMEM((NSLOT, CI, 128), jnp.float32),
        pltpu.VMEM((NSLOT, CI, 128), jnp.float32),
        pltpu.VMEM((NSLOT, CI, 128), jnp.float32),
        pltpu.VMEM((BPW,), jnp.float32),
        pltpu.SemaphoreType.DMA((NSLOT,)),
    ],
)(_body)


def kernel(x, u_embedding, m_embedding, u_bias, m_bias):
    xi = x.T.astype(jnp.int32).reshape(2, NW, BPW)
    big = jnp.concatenate(
        [u_embedding.reshape(-1), m_embedding.reshape(-1),
         u_bias.reshape(-1), m_bias.reshape(-1)]).reshape(-1, 128)
    return _run(big, xi)


# speedup vs baseline: 2.5260x; 2.5260x over previous
import functools

import jax
import jax.numpy as jnp
from jax import lax
from jax.experimental import pallas as pl
from jax.experimental.pallas import tpu as pltpu
from jax.experimental.pallas import tpu_sc as plsc

B = 16384
K = 32
N = 1000000
NC = 2
NS = 16
NW = NC * NS
BPW = B // NW
CI = 64
NCH = BPW // CI
LANES = 16
GPC = CI // LANES
NSLOT = 2


def _body(t4, bias2, xi, out,
          uf_v, mf_v, urow_v, mrow_v, ubrow_v, mbrow_v,
          ubuf, mbuf, ubbuf, mbbuf, out_v, sems):
    wid = lax.axis_index("s") * NC + lax.axis_index("c")

    pltpu.sync_copy(xi.at[0, wid], uf_v)
    pltpu.sync_copy(xi.at[1, wid], mf_v)

    for t in range(BPW // LANES):
        o = t * LANES
        u = uf_v[pl.ds(o, LANES)]
        m = mf_v[pl.ds(o, LANES)]
        urow_v[pl.ds(o, LANES)] = u >> 2
        mrow_v[pl.ds(o, LANES)] = (N // 4) + (m >> 2)
        ubrow_v[pl.ds(o, LANES)] = u >> 7
        mbrow_v[pl.ds(o, LANES)] = (m + N) >> 7

    def fire(j, s):
        sl = pl.ds(j * CI, CI)
        pltpu.async_copy(t4.at[urow_v.at[sl]], ubuf.at[s], sems.at[s])
        pltpu.async_copy(t4.at[mrow_v.at[sl]], mbuf.at[s], sems.at[s])
        pltpu.async_copy(bias2.at[ubrow_v.at[sl]], ubbuf.at[s], sems.at[s])
        pltpu.async_copy(bias2.at[mbrow_v.at[sl]], mbbuf.at[s], sems.at[s])

    def drain(j, s):
        sl = pl.ds(j * CI, CI)
        pltpu.make_async_copy(t4.at[urow_v.at[sl]], ubuf.at[s], sems.at[s]).wait()
        pltpu.make_async_copy(t4.at[mrow_v.at[sl]], mbuf.at[s], sems.at[s]).wait()
        pltpu.make_async_copy(bias2.at[ubrow_v.at[sl]], ubbuf.at[s], sems.at[s]).wait()
        pltpu.make_async_copy(bias2.at[mbrow_v.at[sl]], mbbuf.at[s], sems.at[s]).wait()

    fire(0, 0)
    for j in range(NCH):
        s = j % NSLOT
        drain(j, s)
        if j + 1 < NCH:
            fire(j + 1, (j + 1) % NSLOT)
        for g in range(GPC):
            o = j * CI + g * LANES
            u = uf_v[pl.ds(o, LANES)]
            m = mf_v[pl.ds(o, LANES)]
            lane = g * LANES + lax.iota(jnp.int32, LANES)
            offu = (u & 3) * 32
            offm = (m & 3) * 32
            acc = (plsc.load_gather(ubbuf.at[s], [lane, u & 127]) +
                   plsc.load_gather(mbbuf.at[s], [lane, (m + N) & 127]))
            for k in range(K):
                gu = plsc.load_gather(ubuf.at[s], [lane, offu + k])
                gm = plsc.load_gather(mbuf.at[s], [lane, offm + k])
                acc = acc + gu * gm
            out_v[pl.ds(o, LANES)] = acc

    pltpu.sync_copy(out_v, out.at[pl.ds(wid * BPW, BPW)])


_run = functools.partial(
    pl.kernel,
    out_type=jax.ShapeDtypeStruct((B,), jnp.float32),
    mesh=plsc.VectorSubcoreMesh(core_axis_name="c", subcore_axis_name="s"),
    compiler_params=pltpu.CompilerParams(needs_layout_passes=False),
    scratch_types=[
        pltpu.VMEM((BPW,), jnp.int32),
        pltpu.VMEM((BPW,), jnp.int32),
        pltpu.VMEM((BPW,), jnp.int32),
        pltpu.VMEM((BPW,), jnp.int32),
        pltpu.VMEM((BPW,), jnp.int32),
        pltpu.VMEM((BPW,), jnp.int32),
        pltpu.VMEM((NSLOT, CI, 128), jnp.float32),
        pltpu.VMEM((NSLOT, CI, 128), jnp.float32),
        pltpu.VMEM((NSLOT, CI, 128), jnp.float32),
        pltpu.VMEM((NSLOT, CI, 128), jnp.float32),
        pltpu.VMEM((BPW,), jnp.float32),
        pltpu.SemaphoreType.DMA((NSLOT,)),
    ],
)(_body)


def kernel(x, u_embedding, m_embedding, u_bias, m_bias):
    xi = x.T.astype(jnp.int32).reshape(2, NW, BPW)
    t4 = jnp.concatenate([u_embedding, m_embedding], axis=0).reshape(N // 2, 128)
    bias2 = jnp.concatenate([u_bias, m_bias], axis=0).reshape(2 * N // 128, 128)
    return _run(t4, bias2, xi)


# speedup vs baseline: 3.2538x; 1.2881x over previous
import functools

import jax
import jax.numpy as jnp
from jax import lax
from jax.experimental import pallas as pl
from jax.experimental.pallas import tpu as pltpu
from jax.experimental.pallas import tpu_sc as plsc

B = 16384
K = 32
N = 1000000
NC = 2
NS = 16
NW = NC * NS
BPW = B // NW
CI = 64
NCH = BPW // CI
LANES = 16
GPC = CI // LANES
NSLOT = 2


def _body(u4, m4, bias2, xi, out,
          uf_v, mf_v, urow_v, mrow_v, ubrow_v, mbrow_v,
          ubuf, mbuf, ubbuf, mbbuf, out_v, sems):
    wid = lax.axis_index("s") * NC + lax.axis_index("c")

    pltpu.sync_copy(xi.at[0, wid], uf_v)
    pltpu.sync_copy(xi.at[1, wid], mf_v)

    for t in range(BPW // LANES):
        o = t * LANES
        u = uf_v[pl.ds(o, LANES)]
        m = mf_v[pl.ds(o, LANES)]
        urow_v[pl.ds(o, LANES)] = u >> 2
        mrow_v[pl.ds(o, LANES)] = m >> 2
        ubrow_v[pl.ds(o, LANES)] = u >> 7
        mbrow_v[pl.ds(o, LANES)] = (m + N) >> 7

    def fire(j, s):
        sl = pl.ds(j * CI, CI)
        pltpu.async_copy(u4.at[urow_v.at[sl]], ubuf.at[s], sems.at[s])
        pltpu.async_copy(m4.at[mrow_v.at[sl]], mbuf.at[s], sems.at[s])
        pltpu.async_copy(bias2.at[ubrow_v.at[sl]], ubbuf.at[s], sems.at[s])
        pltpu.async_copy(bias2.at[mbrow_v.at[sl]], mbbuf.at[s], sems.at[s])

    def drain(j, s):
        sl = pl.ds(j * CI, CI)
        pltpu.make_async_copy(u4.at[urow_v.at[sl]], ubuf.at[s], sems.at[s]).wait()
        pltpu.make_async_copy(m4.at[mrow_v.at[sl]], mbuf.at[s], sems.at[s]).wait()
        pltpu.make_async_copy(bias2.at[ubrow_v.at[sl]], ubbuf.at[s], sems.at[s]).wait()
        pltpu.make_async_copy(bias2.at[mbrow_v.at[sl]], mbbuf.at[s], sems.at[s]).wait()

    fire(0, 0)
    for j in range(NCH):
        s = j % NSLOT
        drain(j, s)
        if j + 1 < NCH:
            fire(j + 1, (j + 1) % NSLOT)
        for g in range(GPC):
            o = j * CI + g * LANES
            u = uf_v[pl.ds(o, LANES)]
            m = mf_v[pl.ds(o, LANES)]
            lane = g * LANES + lax.iota(jnp.int32, LANES)
            offu = (u & 3) * 32
            offm = (m & 3) * 32
            acc = (plsc.load_gather(ubbuf.at[s], [lane, u & 127]) +
                   plsc.load_gather(mbbuf.at[s], [lane, (m + N) & 127]))
            for k in range(K):
                gu = plsc.load_gather(ubuf.at[s], [lane, offu + k])
                gm = plsc.load_gather(mbuf.at[s], [lane, offm + k])
                acc = acc + gu * gm
            out_v[pl.ds(o, LANES)] = acc

    pltpu.sync_copy(out_v, out.at[pl.ds(wid * BPW, BPW)])


_run = functools.partial(
    pl.kernel,
    out_type=jax.ShapeDtypeStruct((B,), jnp.float32),
    mesh=plsc.VectorSubcoreMesh(core_axis_name="c", subcore_axis_name="s"),
    compiler_params=pltpu.CompilerParams(needs_layout_passes=False),
    scratch_types=[
        pltpu.VMEM((BPW,), jnp.int32),
        pltpu.VMEM((BPW,), jnp.int32),
        pltpu.VMEM((BPW,), jnp.int32),
        pltpu.VMEM((BPW,), jnp.int32),
        pltpu.VMEM((BPW,), jnp.int32),
        pltpu.VMEM((BPW,), jnp.int32),
        pltpu.VMEM((NSLOT, CI, 128), jnp.float32),
        pltpu.VMEM((NSLOT, CI, 128), jnp.float32),
        pltpu.VMEM((NSLOT, CI, 128), jnp.float32),
        pltpu.VMEM((NSLOT, CI, 128), jnp.float32),
        pltpu.VMEM((BPW,), jnp.float32),
        pltpu.SemaphoreType.DMA((NSLOT,)),
    ],
)(_body)


def kernel(x, u_embedding, m_embedding, u_bias, m_bias):
    xi = x.T.astype(jnp.int32).reshape(2, NW, BPW)
    u4 = u_embedding.reshape(N // 4, 128)
    m4 = m_embedding.reshape(N // 4, 128)
    bias2 = jnp.concatenate([u_bias, m_bias], axis=0).reshape(2 * N // 128, 128)
    return _run(u4, m4, bias2, xi)
